# baseline (speedup 1.0000x reference)
import functools

import jax
import jax.numpy as jnp
from jax import lax
from jax.experimental import pallas as pl
from jax.experimental.pallas import tpu as pltpu
from jax.experimental.pallas import tpu_sc as plsc

N = 10000
E = 320000
D = 128
NC = 2
NS = 16
NW = NC * NS
EPW = E // NW
CH = 80
NCHUNK = EPW // CH
RSUB = 624
TAIL = 16
TAIL0 = NS * RSUB
ZR = 24
DW = 128
LANES = 16

_mesh = plsc.VectorSubcoreMesh(core_axis_name="c", subcore_axis_name="s",
                               num_cores=NC, num_subcores=NS)


def _hist_body(src_hbm, dst_hbm, hist_hbm,
               sidx_v, didx_v, ones_v, zrow_v, acc_sh, sem_a, sem_b):
    c = lax.axis_index("c")
    s = lax.axis_index("s")
    wid = s * NC + c
    r0 = s * RSUB

    pltpu.sync_copy(src_hbm.at[wid], sidx_v)
    pltpu.sync_copy(dst_hbm.at[wid], didx_v)

    def fill_ones(i, carry):
        def fill_col(j, carry2):
            ones_v[i, pl.ds(j * LANES, LANES)] = jnp.full(
                (LANES,), 1.0, jnp.float32)
            return carry2

        lax.fori_loop(0, DW // LANES, fill_col, 0)
        return carry

    lax.fori_loop(0, CH, fill_ones, 0)

    def fill_zero(i, carry):
        def fill_col(j, carry2):
            zrow_v[i, pl.ds(j * LANES, LANES)] = jnp.zeros(
                (LANES,), jnp.float32)
            return carry2

        lax.fori_loop(0, DW // LANES, fill_col, 0)
        return carry

    lax.fori_loop(0, ZR, fill_zero, 0)

    def zero_acc():
        for k in range(RSUB // ZR):
            pltpu.sync_copy(zrow_v, acc_sh.at[pl.ds(r0 + k * ZR, ZR)])

        @pl.when(s == 0)
        def _zero_tail():
            pltpu.sync_copy(zrow_v.at[pl.ds(0, TAIL)],
                            acc_sh.at[pl.ds(TAIL0, TAIL)])

    def scatter_pass(idx_v):
        pltpu.async_copy(ones_v, acc_sh.at[idx_v.at[0]], sem_a, add=True)

        def wait_a():
            pltpu.make_async_copy(ones_v, acc_sh.at[idx_v.at[0]], sem_a).wait()

        def body_loop(k, carry):
            c0 = 2 * k
            db = pltpu.async_copy(ones_v, acc_sh.at[idx_v.at[c0 + 1]],
                                  sem_b, add=True)
            wait_a()
            pltpu.async_copy(ones_v, acc_sh.at[idx_v.at[c0 + 2]],
                             sem_a, add=True)
            db.wait()
            return carry

        lax.fori_loop(0, NCHUNK // 2, body_loop, 0)
        wait_a()

    def copy_out(h):
        pltpu.sync_copy(acc_sh.at[pl.ds(r0, RSUB)],
                        hist_hbm.at[pl.ds(h * NC * N + c * N + r0, RSUB)])

        @pl.when(s == 0)
        def _out_tail():
            pltpu.sync_copy(acc_sh.at[pl.ds(TAIL0, TAIL)],
                            hist_hbm.at[pl.ds(h * NC * N + c * N + TAIL0, TAIL)])

    zero_acc()
    plsc.subcore_barrier()
    scatter_pass(sidx_v)
    plsc.subcore_barrier()
    copy_out(0)
    zero_acc()
    plsc.subcore_barrier()
    scatter_pass(didx_v)
    plsc.subcore_barrier()
    copy_out(1)


_hist_kernel = pl.kernel(
    _hist_body,
    out_type=jax.ShapeDtypeStruct((2 * NC * N, DW), jnp.float32),
    mesh=_mesh,
    scratch_types=[
        pltpu.VMEM((NCHUNK, CH), jnp.int32),
        pltpu.VMEM((NCHUNK, CH), jnp.int32),
        pltpu.VMEM((CH, DW), jnp.float32),
        pltpu.VMEM((ZR, DW), jnp.float32),
        pltpu.VMEM_SHARED((N, DW), jnp.float32),
        pltpu.SemaphoreType.DMA,
        pltpu.SemaphoreType.DMA,
    ],
)


NSLOT = 3


_AGG_OUT = jax.ShapeDtypeStruct((NC * N, D), jnp.float32)
_AGG_SCRATCH = (
    [pltpu.VMEM((EPW,), jnp.int32),
     pltpu.VMEM((NSLOT, CH), jnp.int32)]
    + [pltpu.VMEM((CH, D), jnp.float32) for _ in range(NSLOT)]
    + [pltpu.VMEM_SHARED((N, D), jnp.float32)]
    + [pltpu.SemaphoreType.DMA for _ in range(3 * NSLOT)]
)


def _agg_body(y_hbm, src_hbm, dst_hbm, out_hbm,
              sidx_v, didx_v, *rest):
    rows = rest[:NSLOT]
    acc_sh = rest[NSLOT]
    sem_i = rest[NSLOT + 1:NSLOT + 1 + NSLOT]
    sem_g = rest[NSLOT + 1 + NSLOT:NSLOT + 1 + 2 * NSLOT]
    sem_s = rest[NSLOT + 1 + 2 * NSLOT:]
    c = lax.axis_index("c")
    s = lax.axis_index("s")
    wid = s * NC + c
    r0 = s * RSUB

    pltpu.sync_copy(src_hbm.at[pl.ds(wid * EPW, EPW)], sidx_v)

    def fill_zero(i, carry):
        def fill_row(j, carry2):
            rows[0][i, pl.ds(j * LANES, LANES)] = jnp.zeros((LANES,), jnp.float32)
            return carry2

        lax.fori_loop(0, D // LANES, fill_row, 0)
        return carry

    lax.fori_loop(0, CH, fill_zero, 0)
    for k in range(RSUB // CH):
        pltpu.sync_copy(rows[0], acc_sh.at[pl.ds(r0 + k * CH, CH)])
    _rem = RSUB - (RSUB // CH) * CH
    if _rem:
        pltpu.sync_copy(rows[0].at[pl.ds(0, _rem)],
                        acc_sh.at[pl.ds(r0 + (RSUB // CH) * CH, _rem)])

    @pl.when(s == 0)
    def _zero_tail():
        pltpu.sync_copy(rows[0].at[pl.ds(0, TAIL)], acc_sh.at[pl.ds(TAIL0, TAIL)])

    def issue(ci, j):
        pltpu.async_copy(dst_hbm.at[wid, ci], didx_v.at[j], sem_i[j])
        pltpu.async_copy(y_hbm.at[sidx_v.at[pl.ds(ci * CH, CH)]],
                         rows[j], sem_g[j])

    def wait_issue(j):
        pltpu.make_async_copy(dst_hbm.at[wid, 0], didx_v.at[j], sem_i[j]).wait()
        pltpu.make_async_copy(y_hbm.at[sidx_v.at[pl.ds(0, CH)]],
                              rows[j], sem_g[j]).wait()

    def wait_scatter(j):
        pltpu.make_async_copy(rows[j], acc_sh.at[didx_v.at[j]], sem_s[j]).wait()

    plsc.subcore_barrier()
    for j in range(NSLOT):
        issue(j, j)

    NITER = (NCHUNK + NSLOT - 1) // NSLOT

    def body(k, carry):
        c0 = NSLOT * k
        for j in range(NSLOT):
            @pl.when(c0 + j < NCHUNK)
            def _flush(j=j):
                wait_issue(j)
                pltpu.async_copy(rows[j], acc_sh.at[didx_v.at[j]],
                                 sem_s[j], add=True)

        for j in range(NSLOT):
            @pl.when(c0 + j + NSLOT < NCHUNK)
            def _refill(j=j):
                wait_scatter(j)
                issue(c0 + j + NSLOT, j)

        return carry

    lax.fori_loop(0, NITER, body, 0)
    for j in range(NSLOT):
        wait_scatter(j)
    plsc.subcore_barrier()

    pltpu.sync_copy(acc_sh.at[pl.ds(r0, RSUB)],
                    out_hbm.at[pl.ds(c * N + r0, RSUB)])

    @pl.when(s == 0)
    def _out_tail():
        pltpu.sync_copy(acc_sh.at[pl.ds(TAIL0, TAIL)],
                        out_hbm.at[pl.ds(c * N + TAIL0, TAIL)])


_agg_kernel = pl.kernel(
    _agg_body, out_type=_AGG_OUT, mesh=_mesh, scratch_types=_AGG_SCRATCH)


BLK = 1000
_GRID = N // BLK


def _norm_col(deg_parts):
    deg = deg_parts[0, :, 0:1] + deg_parts[1, :, 0:1]
    return lax.rsqrt(jnp.maximum(deg, 1.0))


def _tc_first_body(x_ref, ds_ref, w_ref, o_ref):
    ns = _norm_col(ds_ref[...])
    o_ref[...] = jnp.dot(x_ref[...] * ns, w_ref[...],
                         preferred_element_type=jnp.float32)


def _tc_mid_body(ap_ref, ds_ref, dd_ref, b_ref, w_ref, o_ref):
    a = ap_ref[0] + ap_ref[1]
    nd = _norm_col(dd_ref[...])
    h = jnp.maximum(a * nd + b_ref[...], 0.0)
    ns = _norm_col(ds_ref[...])
    o_ref[...] = jnp.dot(h * ns, w_ref[...],
                         preferred_element_type=jnp.float32)


def _tc_last_body(ap_ref, dd_ref, b_ref, o_ref):
    a = ap_ref[0] + ap_ref[1]
    nd = _norm_col(dd_ref[...])
    o_ref[...] = a * nd + b_ref[...]


_spec_rows = pl.BlockSpec((BLK, D), lambda i: (i, 0))
_spec_parts = pl.BlockSpec((2, BLK, D), lambda i: (0, i, 0))
_spec_deg = pl.BlockSpec((2, BLK, DW), lambda i: (0, i, 0))
_spec_w = pl.BlockSpec((D, D), lambda i: (0, 0))
_spec_b = pl.BlockSpec((1, D), lambda i: (0, 0))
_out_rows = jax.ShapeDtypeStruct((N, D), jnp.float32)


def _tc_first(x, deg_s, w):
    return pl.pallas_call(
        _tc_first_body,
        grid=(_GRID,),
        in_specs=[_spec_rows, _spec_deg, _spec_w],
        out_specs=_spec_rows,
        out_shape=_out_rows,
    )(x, deg_s, w)


def _tc_mid(agg_parts, deg_s, deg_d, b, w):
    return pl.pallas_call(
        _tc_mid_body,
        grid=(_GRID,),
        in_specs=[_spec_parts, _spec_deg, _spec_deg, _spec_b, _spec_w],
        out_specs=_spec_rows,
        out_shape=_out_rows,
    )(agg_parts, deg_s, deg_d, b, w)


def _tc_last(agg_parts, deg_d, b):
    return pl.pallas_call(
        _tc_last_body,
        grid=(_GRID,),
        in_specs=[_spec_parts, _spec_deg, _spec_b],
        out_specs=_spec_rows,
        out_shape=_out_rows,
    )(agg_parts, deg_d, b)


def kernel(features, edge_index, W0, b0, W1, b1, W2, b2):
    src = edge_index[0]
    src3 = src.reshape(NW, NCHUNK, CH)
    dst3 = edge_index[1].reshape(NW, NCHUNK, CH)

    hists = _hist_kernel(src3, dst3).reshape(2, NC, N, DW)
    deg_s = hists[0]
    deg_d = hists[1]

    t0 = _tc_first(features, deg_s, W0)
    a0 = _agg_kernel(t0, src, dst3).reshape(NC, N, D)
    t1 = _tc_mid(a0, deg_s, deg_d, b0.reshape(1, D), W1)
    a1 = _agg_kernel(t1, src, dst3).reshape(NC, N, D)
    t2 = _tc_mid(a1, deg_s, deg_d, b1.reshape(1, D), W2)
    a2 = _agg_kernel(t2, src, dst3).reshape(NC, N, D)
    return _tc_last(a2, deg_d, b2.reshape(1, D))
